# baseline (speedup 1.0000x reference)
import functools

import jax
import jax.numpy as jnp
from jax import lax
from jax.experimental import pallas as pl
from jax.experimental.pallas import tpu as pltpu
from jax.experimental.pallas import tpu_sc as plsc

N = 10000
E = 320000
D = 128
NC = 2
NS = 16
NW = NC * NS
EW = E // NW
CH = 96
NCHUNK = 112
EWP = NCHUNK * CH
NP = 10112
RT = NP // NS

_mesh = plsc.VectorSubcoreMesh(core_axis_name="c", subcore_axis_name="s")


@functools.partial(
    pl.kernel,
    mesh=_mesh,
    out_type=jax.ShapeDtypeStruct((NC, 3, NP, D), jnp.float32),
    scratch_types=[
        pltpu.VMEM((NCHUNK * CH,), jnp.int32),
        pltpu.VMEM((3, CH), jnp.int32),
        pltpu.VMEM((CH, D), jnp.float32),
        pltpu.VMEM((CH, D), jnp.float32),
        pltpu.VMEM((CH, D), jnp.float32),
        pltpu.VMEM_SHARED((NP, D), jnp.float32),
        pltpu.SemaphoreType.DMA,
        pltpu.SemaphoreType.DMA,
        pltpu.SemaphoreType.DMA,
        pltpu.SemaphoreType.DMA,
        pltpu.SemaphoreType.DMA,
        pltpu.SemaphoreType.DMA,
        pltpu.SemaphoreType.DMA,
        pltpu.SemaphoreType.DMA,
        pltpu.SemaphoreType.DMA,
    ],
)
def _segsum3(tab0_hbm, tab1_hbm, tab2_hbm, src_hbm, dst_hbm, z_hbm, out_hbm,
             src2_v, dstb_v, rows_0, rows_1, rows_2, acc,
             gs0, gs1, gs2, ds0, ds1, ds2, ss0, ss1, ss2):
    c = lax.axis_index("c")
    s = lax.axis_index("s")
    w = c * NS + s
    r0 = s * RT
    dbase = w * EWP

    pltpu.sync_copy(src_hbm.at[pl.ds(w * EWP, EWP)], src2_v)

    rows = (rows_0, rows_1, rows_2)
    gsem = (gs0, gs1, gs2)
    dsem = (ds0, ds1, ds2)
    ssem = (ss0, ss1, ss2)

    def dfire(j, b):
        pltpu.async_copy(dst_hbm.at[pl.ds(dbase + j * CH, CH)],
                         dstb_v.at[b], dsem[b])

    def dwait(b):
        pltpu.make_async_copy(dst_hbm.at[pl.ds(0, CH)],
                              dstb_v.at[b], dsem[b]).wait()

    def sfire(b):
        pltpu.async_copy(rows[b], acc.at[dstb_v.at[b]], ssem[b], add=True)

    def swait(b):
        pltpu.make_async_copy(rows[b], acc.at[dstb_v.at[b]], ssem[b]).wait()

    for t, tab_hbm in enumerate((tab0_hbm, tab1_hbm, tab2_hbm)):
        pltpu.sync_copy(z_hbm.at[pl.ds(r0, RT)], acc.at[pl.ds(r0, RT)])
        plsc.subcore_barrier()

        def gfire(j, b, tab_hbm=tab_hbm):
            pltpu.async_copy(tab_hbm.at[src2_v.at[pl.ds(j * CH, CH)]],
                             rows[b], gsem[b])

        def gwait(b, tab_hbm=tab_hbm):
            pltpu.make_async_copy(tab_hbm.at[src2_v.at[pl.ds(0, CH)]],
                                  rows[b], gsem[b]).wait()

        def body(j, b, b1, first=False, last=False):
            gwait(b)
            dwait(b)
            sfire(b)
            if not first:
                swait(b1)
            if not last:
                gfire(j + 1, b1)
                dfire(j + 1, b1)

        gfire(0, 0)
        dfire(0, 0)
        body(0, 0, 1, first=True)
        body(1, 1, 2, first=True)

        def group(g, carry):
            for r in range(3):
                j = 3 * g + 2 + r
                body(j, (2 + r) % 3, (3 + r) % 3)
            return carry

        lax.fori_loop(0, (NCHUNK - 4) // 3, group, 0)
        body(NCHUNK - 2, (NCHUNK - 2) % 3, (NCHUNK - 1) % 3)
        body(NCHUNK - 1, (NCHUNK - 1) % 3, NCHUNK % 3, last=True)
        swait((NCHUNK - 2) % 3)
        swait((NCHUNK - 1) % 3)

        plsc.subcore_barrier()
        pltpu.sync_copy(acc.at[pl.ds(r0, RT)],
                        out_hbm.at[c, t, pl.ds(r0, RT)])
        plsc.subcore_barrier()


_BR = 1000


def _prep_body(x_ref, age_ref, out_ref):
    x = x_ref[...]
    age = age_ref[...]
    m1 = (age >= 1).astype(jnp.float32)
    m2 = (age >= 2).astype(jnp.float32)
    out_ref[...] = jnp.stack([x, x * m1, x * m2])


def _prep(x, age2d):
    return pl.pallas_call(
        _prep_body,
        grid=(N // _BR,),
        in_specs=[
            pl.BlockSpec((_BR, D), lambda i: (i, 0)),
            pl.BlockSpec((_BR, 1), lambda i: (i, 0)),
        ],
        out_specs=pl.BlockSpec((3, _BR, D), lambda i: (0, i, 0)),
        out_shape=jax.ShapeDtypeStruct((3, N, D), jnp.float32),
    )(x, age2d)


def _combine_body(p_ref, h1_ref):
    p = p_ref[...]
    h1_ref[...] = jnp.maximum(p[0] + p[1], 0.0)


def _combine(part):
    return pl.pallas_call(
        _combine_body,
        grid=(N // _BR,),
        in_specs=[pl.BlockSpec((NC, 3, _BR, D), lambda i: (0, 0, i, 0))],
        out_specs=pl.BlockSpec((3, _BR, D), lambda i: (0, i, 0)),
        out_shape=jax.ShapeDtypeStruct((3, N, D), jnp.float32),
    )(part)


def _final_body(x_ref, age_ref, h1_ref, p2_ref, out_ref):
    x = x_ref[...]
    age = age_ref[...]
    h1 = h1_ref[...]
    p2 = p2_ref[...]
    h2 = jnp.maximum(p2[0] + p2[1], 0.0)
    m1 = (age >= 1).astype(jnp.float32)
    m2 = (age >= 2).astype(jnp.float32)
    out_ref[...] = jnp.concatenate([
        x,
        h1[0],
        h2[0],
        x * (1.0 - 0.5 * (m1 + m2)),
        h1[0] - 0.5 * (h1[1] + h1[2]),
        h2[0] - 0.5 * (h2[1] + h2[2]),
    ], axis=1)


def _final(x, age2d, h1, part2):
    return pl.pallas_call(
        _final_body,
        grid=(N // _BR,),
        in_specs=[
            pl.BlockSpec((_BR, D), lambda i: (i, 0)),
            pl.BlockSpec((_BR, 1), lambda i: (i, 0)),
            pl.BlockSpec((3, _BR, D), lambda i: (0, i, 0)),
            pl.BlockSpec((NC, 3, _BR, D), lambda i: (0, 0, i, 0)),
        ],
        out_specs=pl.BlockSpec((_BR, 6 * D), lambda i: (i, 0)),
        out_shape=jax.ShapeDtypeStruct((N, 6 * D), jnp.float32),
    )(x, age2d, h1, part2)


def kernel(x, age, edge_index):
    src = edge_index[0]
    dst = edge_index[1]
    age2d = age[:, None]
    z = jnp.zeros((NP, D), jnp.float32)

    npad = EWP - EW
    pad_src = jnp.broadcast_to(jnp.arange(npad, dtype=jnp.int32), (NW, npad))
    pad_dst = jnp.broadcast_to(
        N + (jnp.arange(npad, dtype=jnp.int32) % (NP - N)), (NW, npad))
    srcp = jnp.concatenate([src.reshape(NW, EW), pad_src], axis=1)
    srcp = srcp.reshape(NW * EWP)
    dstp = jnp.concatenate([dst.reshape(NW, EW), pad_dst], axis=1)
    dstp = dstp.reshape(NW * EWP)

    xt = _prep(x, age2d)
    part1 = _segsum3(xt[0], xt[1], xt[2], srcp, dstp, z)
    h1 = _combine(part1)
    part2 = _segsum3(h1[0], h1[1], h1[2], srcp, dstp, z)
    return _final(x, age2d, h1, part2)


# speedup vs baseline: 1.3552x; 1.3552x over previous
import functools

import jax
import jax.numpy as jnp
from jax import lax
from jax.experimental import pallas as pl
from jax.experimental.pallas import tpu as pltpu
from jax.experimental.pallas import tpu_sc as plsc

N = 10000
E = 320000
D = 128
NC = 2
NS = 16
NW = NC * NS
EW = E // NW
CH = 128
NCHUNK = 80
EWP = NCHUNK * CH
NP = 10112
RT = NP // NS

_mesh = plsc.VectorSubcoreMesh(core_axis_name="c", subcore_axis_name="s")


@functools.partial(
    pl.kernel,
    mesh=_mesh,
    out_type=jax.ShapeDtypeStruct((NC, 3, NP, D), jnp.float32),
    scratch_types=[
        pltpu.VMEM((NCHUNK * CH,), jnp.int32),
        pltpu.VMEM((2, CH), jnp.int32),
        pltpu.VMEM((CH, D), jnp.float32),
        pltpu.VMEM((CH, D), jnp.float32),
        pltpu.VMEM_SHARED((NP, D), jnp.float32),
        pltpu.SemaphoreType.DMA,
        pltpu.SemaphoreType.DMA,
        pltpu.SemaphoreType.DMA,
        pltpu.SemaphoreType.DMA,
    ],
)
def _segsum3(tab0_hbm, tab1_hbm, tab2_hbm, src_hbm, dst_hbm, z_hbm, out_hbm,
             src2_v, dstb_v, rows_a, rows_b, acc, sem_a, sem_b, dsem_a, dsem_b):
    c = lax.axis_index("c")
    s = lax.axis_index("s")
    w = c * NS + s
    r0 = s * RT
    dbase = w * EWP

    pltpu.sync_copy(src_hbm.at[pl.ds(w * EWP, EWP)], src2_v)

    rows = (rows_a, rows_b)
    gsem = (sem_a, sem_b)
    dsem = (dsem_a, dsem_b)

    def dload(j, b):
        pltpu.async_copy(dst_hbm.at[pl.ds(dbase + j * CH, CH)],
                         dstb_v.at[b], dsem[b])

    def dwait(b):
        pltpu.make_async_copy(dst_hbm.at[pl.ds(0, CH)],
                              dstb_v.at[b], dsem[b]).wait()

    for t, tab_hbm in enumerate((tab0_hbm, tab1_hbm, tab2_hbm)):
        pltpu.sync_copy(z_hbm.at[pl.ds(r0, RT)], acc.at[pl.ds(r0, RT)])
        plsc.subcore_barrier()

        def gather(j, b, tab_hbm=tab_hbm):
            pltpu.async_copy(tab_hbm.at[src2_v.at[pl.ds(j * CH, CH)]],
                             rows[b], gsem[b])

        def gwait(b, tab_hbm=tab_hbm):
            pltpu.make_async_copy(tab_hbm.at[src2_v.at[pl.ds(0, CH)]],
                                  rows[b], gsem[b]).wait()

        def scatter(b):
            pltpu.sync_copy(rows[b], acc.at[dstb_v.at[b]], add=True)

        gather(0, 0), dload(0, 0)
        gather(1, 1), dload(1, 1)

        def pair(jj, carry):
            j0 = jj * 2
            for b in range(2):
                gwait(b)
                dwait(b)
                scatter(b)
                gather(j0 + 2 + b, b)
                dload(j0 + 2 + b, b)
            return carry

        lax.fori_loop(0, NCHUNK // 2 - 1, pair, 0)
        for b in range(2):
            gwait(b)
            dwait(b)
            scatter(b)

        plsc.subcore_barrier()
        pltpu.sync_copy(acc.at[pl.ds(r0, RT)],
                        out_hbm.at[c, t, pl.ds(r0, RT)])
        plsc.subcore_barrier()


_BR = 1000


def _prep_body(x_ref, age_ref, out_ref):
    x = x_ref[...]
    age = age_ref[...]
    m1 = (age >= 1).astype(jnp.float32)
    m2 = (age >= 2).astype(jnp.float32)
    out_ref[...] = jnp.stack([x, x * m1, x * m2])


def _prep(x, age2d):
    return pl.pallas_call(
        _prep_body,
        grid=(N // _BR,),
        in_specs=[
            pl.BlockSpec((_BR, D), lambda i: (i, 0)),
            pl.BlockSpec((_BR, 1), lambda i: (i, 0)),
        ],
        out_specs=pl.BlockSpec((3, _BR, D), lambda i: (0, i, 0)),
        out_shape=jax.ShapeDtypeStruct((3, N, D), jnp.float32),
    )(x, age2d)


def _combine_body(p_ref, h1_ref):
    p = p_ref[...]
    h1_ref[...] = jnp.maximum(p[0] + p[1], 0.0)


def _combine(part):
    return pl.pallas_call(
        _combine_body,
        grid=(N // _BR,),
        in_specs=[pl.BlockSpec((NC, 3, _BR, D), lambda i: (0, 0, i, 0))],
        out_specs=pl.BlockSpec((3, _BR, D), lambda i: (0, i, 0)),
        out_shape=jax.ShapeDtypeStruct((3, N, D), jnp.float32),
    )(part)


def _final_body(x_ref, age_ref, h1_ref, p2_ref, out_ref):
    x = x_ref[...]
    age = age_ref[...]
    h1 = h1_ref[...]
    p2 = p2_ref[...]
    h2 = jnp.maximum(p2[0] + p2[1], 0.0)
    m1 = (age >= 1).astype(jnp.float32)
    m2 = (age >= 2).astype(jnp.float32)
    out_ref[...] = jnp.concatenate([
        x,
        h1[0],
        h2[0],
        x * (1.0 - 0.5 * (m1 + m2)),
        h1[0] - 0.5 * (h1[1] + h1[2]),
        h2[0] - 0.5 * (h2[1] + h2[2]),
    ], axis=1)


def _final(x, age2d, h1, part2):
    return pl.pallas_call(
        _final_body,
        grid=(N // _BR,),
        in_specs=[
            pl.BlockSpec((_BR, D), lambda i: (i, 0)),
            pl.BlockSpec((_BR, 1), lambda i: (i, 0)),
            pl.BlockSpec((3, _BR, D), lambda i: (0, i, 0)),
            pl.BlockSpec((NC, 3, _BR, D), lambda i: (0, 0, i, 0)),
        ],
        out_specs=pl.BlockSpec((_BR, 6 * D), lambda i: (i, 0)),
        out_shape=jax.ShapeDtypeStruct((N, 6 * D), jnp.float32),
    )(x, age2d, h1, part2)


def kernel(x, age, edge_index):
    src = edge_index[0]
    dst = edge_index[1]
    age2d = age[:, None]
    z = jnp.zeros((NP, D), jnp.float32)

    npad = EWP - EW
    pad_src = jnp.broadcast_to(jnp.arange(npad, dtype=jnp.int32), (NW, npad))
    pad_dst = jnp.broadcast_to(
        N + (jnp.arange(npad, dtype=jnp.int32) % (NP - N)), (NW, npad))
    srcp = jnp.concatenate([src.reshape(NW, EW), pad_src], axis=1)
    srcp = srcp.reshape(NW * EWP)
    dstp = jnp.concatenate([dst.reshape(NW, EW), pad_dst], axis=1)
    dstp = dstp.reshape(NW * EWP)

    xt = _prep(x, age2d)
    part1 = _segsum3(xt[0], xt[1], xt[2], srcp, dstp, z)
    h1 = _combine(part1)
    part2 = _segsum3(h1[0], h1[1], h1[2], srcp, dstp, z)
    return _final(x, age2d, h1, part2)


# speedup vs baseline: 1.3634x; 1.0061x over previous
import functools

import jax
import jax.numpy as jnp
from jax import lax
from jax.experimental import pallas as pl
from jax.experimental.pallas import tpu as pltpu
from jax.experimental.pallas import tpu_sc as plsc

N = 10000
E = 320000
D = 128
NC = 2
NS = 16
NW = NC * NS
EW = E // NW
CH = 128
NCHUNK = 79
EWP = NCHUNK * CH
NP = 10112
RT = NP // NS

_mesh = plsc.VectorSubcoreMesh(core_axis_name="c", subcore_axis_name="s")


@functools.partial(
    pl.kernel,
    mesh=_mesh,
    out_type=jax.ShapeDtypeStruct((NC, 3, NP, D), jnp.float32),
    scratch_types=[
        pltpu.VMEM((NCHUNK * CH,), jnp.int32),
        pltpu.VMEM((2, CH), jnp.int32),
        pltpu.VMEM((CH, D), jnp.float32),
        pltpu.VMEM((CH, D), jnp.float32),
        pltpu.VMEM_SHARED((NP, D), jnp.float32),
        pltpu.SemaphoreType.DMA,
        pltpu.SemaphoreType.DMA,
        pltpu.SemaphoreType.DMA,
        pltpu.SemaphoreType.DMA,
    ],
)
def _segsum3(tab0_hbm, tab1_hbm, tab2_hbm, src_hbm, dst_hbm, z_hbm, out_hbm,
             src2_v, dstb_v, rows_a, rows_b, acc, sem_a, sem_b, dsem_a, dsem_b):
    c = lax.axis_index("c")
    s = lax.axis_index("s")
    w = c * NS + s
    r0 = s * RT
    dbase = w * EWP

    pltpu.sync_copy(src_hbm.at[pl.ds(w * EWP, EWP)], src2_v)

    rows = (rows_a, rows_b)
    gsem = (sem_a, sem_b)
    dsem = (dsem_a, dsem_b)

    def dload(j, b):
        pltpu.async_copy(dst_hbm.at[pl.ds(dbase + j * CH, CH)],
                         dstb_v.at[b], dsem[b])

    def dwait(b):
        pltpu.make_async_copy(dst_hbm.at[pl.ds(0, CH)],
                              dstb_v.at[b], dsem[b]).wait()

    for t, tab_hbm in enumerate((tab0_hbm, tab1_hbm, tab2_hbm)):
        pltpu.sync_copy(z_hbm.at[pl.ds(r0, RT)], acc.at[pl.ds(r0, RT)])
        plsc.subcore_barrier()

        def gather(j, b, tab_hbm=tab_hbm):
            pltpu.async_copy(tab_hbm.at[src2_v.at[pl.ds(j * CH, CH)]],
                             rows[b], gsem[b])

        def gwait(b, tab_hbm=tab_hbm):
            pltpu.make_async_copy(tab_hbm.at[src2_v.at[pl.ds(0, CH)]],
                                  rows[b], gsem[b]).wait()

        def scatter(b):
            pltpu.sync_copy(rows[b], acc.at[dstb_v.at[b]], add=True)

        gather(0, 0), dload(0, 0)
        gather(1, 1), dload(1, 1)

        def pair(jj, carry):
            j0 = jj * 2
            for b in range(2):
                gwait(b)
                dwait(b)
                scatter(b)
                gather(j0 + 2 + b, b)
                dload(j0 + 2 + b, b)
            return carry

        lax.fori_loop(0, (NCHUNK - 3) // 2, pair, 0)
        gwait(0)
        dwait(0)
        scatter(0)
        gather(NCHUNK - 1, 0)
        dload(NCHUNK - 1, 0)
        gwait(1)
        dwait(1)
        scatter(1)
        gwait(0)
        dwait(0)
        scatter(0)

        plsc.subcore_barrier()
        pltpu.sync_copy(acc.at[pl.ds(r0, RT)],
                        out_hbm.at[c, t, pl.ds(r0, RT)])
        plsc.subcore_barrier()


_BR = 1000


def _prep_body(x_ref, age_ref, out_ref):
    x = x_ref[...]
    age = age_ref[...]
    m1 = (age >= 1).astype(jnp.float32)
    m2 = (age >= 2).astype(jnp.float32)
    out_ref[...] = jnp.stack([x * m1, x * m2])


def _prep(x, age2d):
    return pl.pallas_call(
        _prep_body,
        grid=(N // _BR,),
        in_specs=[
            pl.BlockSpec((_BR, D), lambda i: (i, 0)),
            pl.BlockSpec((_BR, 1), lambda i: (i, 0)),
        ],
        out_specs=pl.BlockSpec((2, _BR, D), lambda i: (0, i, 0)),
        out_shape=jax.ShapeDtypeStruct((2, N, D), jnp.float32),
    )(x, age2d)


def _combine_body(p_ref, h1_ref):
    p = p_ref[...]
    h1_ref[...] = jnp.maximum(p[0] + p[1], 0.0)


def _combine(part):
    return pl.pallas_call(
        _combine_body,
        grid=(N // _BR,),
        in_specs=[pl.BlockSpec((NC, 3, _BR, D), lambda i: (0, 0, i, 0))],
        out_specs=pl.BlockSpec((3, _BR, D), lambda i: (0, i, 0)),
        out_shape=jax.ShapeDtypeStruct((3, N, D), jnp.float32),
    )(part)


def _final_body(x_ref, age_ref, h1_ref, p2_ref, out_ref):
    x = x_ref[...]
    age = age_ref[...]
    h1 = h1_ref[...]
    p2 = p2_ref[...]
    h2 = jnp.maximum(p2[0] + p2[1], 0.0)
    m1 = (age >= 1).astype(jnp.float32)
    m2 = (age >= 2).astype(jnp.float32)
    out_ref[...] = jnp.concatenate([
        x,
        h1[0],
        h2[0],
        x * (1.0 - 0.5 * (m1 + m2)),
        h1[0] - 0.5 * (h1[1] + h1[2]),
        h2[0] - 0.5 * (h2[1] + h2[2]),
    ], axis=1)


def _final(x, age2d, h1, part2):
    return pl.pallas_call(
        _final_body,
        grid=(N // _BR,),
        in_specs=[
            pl.BlockSpec((_BR, D), lambda i: (i, 0)),
            pl.BlockSpec((_BR, 1), lambda i: (i, 0)),
            pl.BlockSpec((3, _BR, D), lambda i: (0, i, 0)),
            pl.BlockSpec((NC, 3, _BR, D), lambda i: (0, 0, i, 0)),
        ],
        out_specs=pl.BlockSpec((_BR, 6 * D), lambda i: (i, 0)),
        out_shape=jax.ShapeDtypeStruct((N, 6 * D), jnp.float32),
    )(x, age2d, h1, part2)


def kernel(x, age, edge_index):
    src = edge_index[0]
    dst = edge_index[1]
    age2d = age[:, None]
    z = jnp.zeros((NP, D), jnp.float32)

    npad = EWP - EW
    pad_src = jnp.broadcast_to(jnp.arange(npad, dtype=jnp.int32), (NW, npad))
    pad_dst = jnp.broadcast_to(
        N + (jnp.arange(npad, dtype=jnp.int32) % (NP - N)), (NW, npad))
    srcp = jnp.concatenate([src.reshape(NW, EW), pad_src], axis=1)
    srcp = srcp.reshape(NW * EWP)
    dstp = jnp.concatenate([dst.reshape(NW, EW), pad_dst], axis=1)
    dstp = dstp.reshape(NW * EWP)

    xt = _prep(x, age2d)
    part1 = _segsum3(x, xt[0], xt[1], srcp, dstp, z)
    h1 = _combine(part1)
    part2 = _segsum3(h1[0], h1[1], h1[2], srcp, dstp, z)
    return _final(x, age2d, h1, part2)
